# baseline (speedup 1.0000x reference)
import jax
import jax.numpy as jnp
from jax.experimental import pallas as pl
from jax.experimental.pallas import tpu as pltpu

N = 128
C, H, W = 3, 224, 224
D = C * H * W
SPB = 8
STEPS = N // SPB
REM = int(N * (1.0 - 0.1))


def _acc_kernel(xr1_ref, xr2_ref, x_ref, acc1_ref, acc2_ref):
    x = x_ref[...]
    d1 = xr1_ref[...] - x
    d2 = xr2_ref[...] - x
    acc1_ref[...] = jnp.sum(d1 * d1, axis=(1, 2, 3), keepdims=True)
    acc2_ref[...] = jnp.sum(d2 * d2, axis=(1, 2, 3), keepdims=True)


def _select_kernel(a1c_ref, a2c_ref, a1r_ref, a2r_ref, l1_ref, l2_ref):
    a1c = a1c_ref[...]
    a2c = a2c_ref[...]
    a1r = a1r_ref[...]
    a2r = a2r_ref[...]
    jidx = jax.lax.broadcasted_iota(jnp.int32, (N, N), 1)
    iidx = jax.lax.broadcasted_iota(jnp.int32, (N, N), 0)
    tie = jidx < iidx
    cmp2 = (a2r < a2c) | ((a2r == a2c) & tie)
    cmp1 = (a1r < a1c) | ((a1r == a1c) & tie)
    rank2 = jnp.sum(cmp2.astype(jnp.int32), axis=1, keepdims=True)
    rank1 = jnp.sum(cmp1.astype(jnp.int32), axis=1, keepdims=True)
    sel2 = rank2 < REM
    sel1 = rank1 < REM
    scale = 1.0 / (REM * D)
    l1_ref[...] = jnp.sum(jnp.where(sel2, a1c, 0.0), axis=0, keepdims=True) * scale
    l2_ref[...] = jnp.sum(jnp.where(sel1, a2c, 0.0), axis=0, keepdims=True) * scale


def kernel(xr1, xr2, x):
    spec = pl.BlockSpec((SPB, C, H, W), lambda i: (i, 0, 0, 0))
    acc_spec = pl.BlockSpec((SPB, 1, 1, 1), lambda i: (i, 0, 0, 0))
    acc1, acc2 = pl.pallas_call(
        _acc_kernel,
        grid=(STEPS,),
        in_specs=[spec, spec, spec],
        out_specs=[acc_spec, acc_spec],
        out_shape=[
            jax.ShapeDtypeStruct((N, 1, 1, 1), jnp.float32),
            jax.ShapeDtypeStruct((N, 1, 1, 1), jnp.float32),
        ],
        compiler_params=pltpu.CompilerParams(
            dimension_semantics=("arbitrary",),
        ),
    )(xr1, xr2, x)

    a1c = acc1.reshape(N, 1)
    a2c = acc2.reshape(N, 1)
    a1r = acc1.reshape(1, N)
    a2r = acc2.reshape(1, N)
    l1, l2 = pl.pallas_call(
        _select_kernel,
        out_shape=[
            jax.ShapeDtypeStruct((1, 1), jnp.float32),
            jax.ShapeDtypeStruct((1, 1), jnp.float32),
        ],
    )(a1c, a2c, a1r, a2r)
    return (l1.reshape(()), l2.reshape(()))


# speedup vs baseline: 1.1613x; 1.1613x over previous
import jax
import jax.numpy as jnp
from jax.experimental import pallas as pl
from jax.experimental.pallas import tpu as pltpu

N = 128
D = 3 * 224 * 224
ROWS = D // 128
SPB = 2
STEPS = N // SPB
NSLOT = 8
REM = int(N * (1.0 - 0.1))


def _acc_kernel(x1_hbm, x2_hbm, xx_hbm, acc1_ref, acc2_ref, b1, b2, bx, sems):
    def streams(s, slot):
        out = []
        for k, (hbm, buf) in enumerate(((x1_hbm, b1), (x2_hbm, b2), (xx_hbm, bx))):
            for h in range(SPB):
                out.append((pltpu.make_async_copy(
                    hbm.at[pl.ds(s * SPB + h, 1)],
                    buf.at[slot, pl.ds(h, 1)],
                    sems.at[k, h, slot]), (SPB * k + h) % 2))
        return out

    def start_copies(s, slot):
        for copy, prio in streams(s, slot):
            copy.start(priority=prio)

    for s in range(NSLOT):
        start_copies(s, s)

    def body(s, _):
        slot = jax.lax.rem(s, NSLOT)
        for copy, _prio in streams(s, slot):
            copy.wait()
        x = bx[slot]
        d1 = b1[slot] - x
        d2 = b2[slot] - x
        acc1_ref[s] = jnp.sum(d1 * d1, axis=(1, 2)).reshape(SPB, 1)
        acc2_ref[s] = jnp.sum(d2 * d2, axis=(1, 2)).reshape(SPB, 1)

        @pl.when(s + NSLOT < STEPS)
        def _():
            start_copies(s + NSLOT, slot)

        return 0

    jax.lax.fori_loop(0, STEPS, body, 0)


def _select_kernel(a1c_ref, a2c_ref, a1r_ref, a2r_ref, l1_ref, l2_ref):
    a1c = a1c_ref[...]
    a2c = a2c_ref[...]
    a1r = a1r_ref[...]
    a2r = a2r_ref[...]
    jidx = jax.lax.broadcasted_iota(jnp.int32, (N, N), 1)
    iidx = jax.lax.broadcasted_iota(jnp.int32, (N, N), 0)
    tie = jidx < iidx
    cmp2 = (a2r < a2c) | ((a2r == a2c) & tie)
    cmp1 = (a1r < a1c) | ((a1r == a1c) & tie)
    rank2 = jnp.sum(cmp2.astype(jnp.int32), axis=1, keepdims=True)
    rank1 = jnp.sum(cmp1.astype(jnp.int32), axis=1, keepdims=True)
    sel2 = rank2 < REM
    sel1 = rank1 < REM
    scale = 1.0 / (REM * D)
    l1_ref[...] = jnp.sum(jnp.where(sel2, a1c, 0.0), axis=0, keepdims=True) * scale
    l2_ref[...] = jnp.sum(jnp.where(sel1, a2c, 0.0), axis=0, keepdims=True) * scale


def kernel(xr1, xr2, x):
    xr1 = xr1.reshape(N, ROWS, 128)
    xr2 = xr2.reshape(N, ROWS, 128)
    x = x.reshape(N, ROWS, 128)

    any_spec = pl.BlockSpec(memory_space=pl.ANY)
    acc1, acc2 = pl.pallas_call(
        _acc_kernel,
        in_specs=[any_spec, any_spec, any_spec],
        out_shape=[
            jax.ShapeDtypeStruct((STEPS, SPB, 1), jnp.float32),
            jax.ShapeDtypeStruct((STEPS, SPB, 1), jnp.float32),
        ],
        scratch_shapes=[
            pltpu.VMEM((NSLOT, SPB, ROWS, 128), jnp.float32),
            pltpu.VMEM((NSLOT, SPB, ROWS, 128), jnp.float32),
            pltpu.VMEM((NSLOT, SPB, ROWS, 128), jnp.float32),
            pltpu.SemaphoreType.DMA((3, SPB, NSLOT)),
        ],
    )(xr1, xr2, x)

    a1c = acc1.reshape(N, 1)
    a2c = acc2.reshape(N, 1)
    a1r = acc1.reshape(1, N)
    a2r = acc2.reshape(1, N)
    l1, l2 = pl.pallas_call(
        _select_kernel,
        out_shape=[
            jax.ShapeDtypeStruct((1, 1), jnp.float32),
            jax.ShapeDtypeStruct((1, 1), jnp.float32),
        ],
    )(a1c, a2c, a1r, a2r)
    return (l1.reshape(()), l2.reshape(()))


# speedup vs baseline: 1.2463x; 1.0732x over previous
import jax
import jax.numpy as jnp
from jax.experimental import pallas as pl
from jax.experimental.pallas import tpu as pltpu

N = 128
D = 3 * 224 * 224
CHUNK = 6272
STEPS = D // CHUNK
REM = int(N * (1.0 - 0.1))


def _acc_kernel(xr1_ref, xr2_ref, x_ref, acc1_ref, acc2_ref):
    i = pl.program_id(0)
    x = x_ref[...]
    d1 = xr1_ref[...] - x
    d2 = xr2_ref[...] - x
    p1 = jnp.sum(d1 * d1, axis=1, keepdims=True)
    p2 = jnp.sum(d2 * d2, axis=1, keepdims=True)

    @pl.when(i == 0)
    def _init():
        acc1_ref[...] = p1
        acc2_ref[...] = p2

    @pl.when(i > 0)
    def _accum():
        acc1_ref[...] += p1
        acc2_ref[...] += p2


def _select_kernel(a1c_ref, a2c_ref, a1r_ref, a2r_ref, l1_ref, l2_ref):
    a1c = a1c_ref[...]
    a2c = a2c_ref[...]
    a1r = a1r_ref[...]
    a2r = a2r_ref[...]
    jidx = jax.lax.broadcasted_iota(jnp.int32, (N, N), 1)
    iidx = jax.lax.broadcasted_iota(jnp.int32, (N, N), 0)
    tie = jidx < iidx
    cmp2 = (a2r < a2c) | ((a2r == a2c) & tie)
    cmp1 = (a1r < a1c) | ((a1r == a1c) & tie)
    rank2 = jnp.sum(cmp2.astype(jnp.int32), axis=1, keepdims=True)
    rank1 = jnp.sum(cmp1.astype(jnp.int32), axis=1, keepdims=True)
    sel2 = rank2 < REM
    sel1 = rank1 < REM
    scale = 1.0 / (REM * D)
    l1_ref[...] = jnp.sum(jnp.where(sel2, a1c, 0.0), axis=0, keepdims=True) * scale
    l2_ref[...] = jnp.sum(jnp.where(sel1, a2c, 0.0), axis=0, keepdims=True) * scale


def kernel(xr1, xr2, x):
    xr1 = xr1.reshape(N, D)
    xr2 = xr2.reshape(N, D)
    x = x.reshape(N, D)

    spec = pl.BlockSpec((N, CHUNK), lambda i: (0, i))
    acc_spec = pl.BlockSpec((N, 1), lambda i: (0, 0))
    acc1, acc2 = pl.pallas_call(
        _acc_kernel,
        grid=(STEPS,),
        in_specs=[spec, spec, spec],
        out_specs=[acc_spec, acc_spec],
        out_shape=[
            jax.ShapeDtypeStruct((N, 1), jnp.float32),
            jax.ShapeDtypeStruct((N, 1), jnp.float32),
        ],
    )(xr1, xr2, x)

    a1r = acc1.reshape(1, N)
    a2r = acc2.reshape(1, N)
    l1, l2 = pl.pallas_call(
        _select_kernel,
        out_shape=[
            jax.ShapeDtypeStruct((1, 1), jnp.float32),
            jax.ShapeDtypeStruct((1, 1), jnp.float32),
        ],
    )(acc1, acc2, a1r, a2r)
    return (l1.reshape(()), l2.reshape(()))
